# Initial kernel scaffold; baseline (speedup 1.0000x reference)
import jax
import jax.numpy as jnp
from jax.experimental import pallas as pl


def kernel(token_ids, embeddings, projection):
    raise NotImplementedError("write your pallas kernel here")


# speedup vs baseline: 2.8548x; 2.8548x over previous
import functools

import jax
import jax.numpy as jnp
from jax import lax
from jax.experimental import pallas as pl
from jax.experimental.pallas import tpu as pltpu
from jax.experimental.pallas import tpu_sc as plsc

_NUM_CORES = 2
_NUM_SUBCORES = 16
_NUM_WORKERS = _NUM_CORES * _NUM_SUBCORES
_LANES = 16

_BATCH = 16384
_SEQ = 50
_DIM = 32
_OUT_DIM = 16

_ROWS_W = _BATCH // _NUM_WORKERS
_CB = 16
_G = _CB * _SEQ
_SUB = 80
_NSUB = _G // _SUB
_NCHUNK = _ROWS_W // _CB
_IDX_W = _ROWS_W * _SEQ

_mesh = plsc.VectorSubcoreMesh(core_axis_name="c", subcore_axis_name="s")


@functools.partial(
    pl.kernel,
    out_type=jax.ShapeDtypeStruct((_BATCH, _DIM), jnp.float32),
    mesh=_mesh,
    scratch_types=[
        pltpu.VMEM((2, _NSUB, _SUB), jnp.int32),
        pltpu.VMEM((2, _G, _DIM), jnp.float32),
        pltpu.VMEM((_ROWS_W, _DIM), jnp.float32),
        pltpu.SemaphoreType.DMA,
        pltpu.SemaphoreType.DMA,
    ],
    compiler_params=pltpu.CompilerParams(use_tc_tiling_on_sc=False),
)
def _gather_pool(idx_hbm, table_hbm, out_hbm, idx_v, rows_v, out_v, sem0, sem1):
    wid = lax.axis_index("s") * _NUM_CORES + lax.axis_index("c")
    ibase = wid * _NCHUNK
    sems = (sem0, sem1)

    def fire(chunk, b):
        pltpu.sync_copy(idx_hbm.at[ibase + chunk], idx_v.at[b])
        for j in range(_NSUB):
            pltpu.async_copy(
                table_hbm.at[idx_v.at[b].at[j]],
                rows_v.at[b].at[pl.ds(j * _SUB, _SUB)],
                sems[b],
            )

    def drain(b):
        pltpu.make_async_copy(
            table_hbm.at[pl.ds(0, _G)], rows_v.at[b], sems[b]
        ).wait()

    def reduce(chunk, b):
        rows = rows_v.at[b]

        @pl.loop(0, _CB)
        def _row(r):
            base = r * _SEQ
            acc_lo = [None] * 4
            acc_hi = [None] * 4
            for t in range(_SEQ):
                row = rows.at[base + t]
                lo = row[pl.ds(0, _LANES)]
                hi = row[pl.ds(_LANES, _LANES)]
                a = t % 4
                acc_lo[a] = lo if acc_lo[a] is None else acc_lo[a] + lo
                acc_hi[a] = hi if acc_hi[a] is None else acc_hi[a] + hi
            orow = chunk * _CB + r
            out_v[orow, pl.ds(0, _LANES)] = (acc_lo[0] + acc_lo[1]) + (acc_lo[2] + acc_lo[3])
            out_v[orow, pl.ds(_LANES, _LANES)] = (acc_hi[0] + acc_hi[1]) + (acc_hi[2] + acc_hi[3])

    fire(0, 0)

    @pl.loop(0, _NCHUNK - 2, step=2)
    def _chunks(c):
        for b in range(2):
            fire(c + b + 1, 1 - b)
            drain(b)
            reduce(c + b, b)

    fire(_NCHUNK - 1, 1)
    drain(0)
    reduce(_NCHUNK - 2, 0)
    drain(1)
    reduce(_NCHUNK - 1, 1)

    pltpu.sync_copy(out_v, out_hbm.at[pl.ds(wid * _ROWS_W, _ROWS_W)])


def _project(pooled, projection):
    def body(x_ref, p_ref, o_ref):
        o_ref[...] = jnp.dot(
            x_ref[...],
            p_ref[...] * (1.0 / _SEQ),
            preferred_element_type=jnp.float32,
        )

    return pl.pallas_call(
        body,
        out_shape=jax.ShapeDtypeStruct((_BATCH, _OUT_DIM), jnp.float32),
    )(pooled, projection)


def kernel(token_ids, embeddings, projection):
    flat = token_ids.astype(jnp.int32).reshape(_NUM_WORKERS * _NCHUNK, _NSUB, _SUB)
    pooled = _gather_pool(flat, embeddings)
    return _project(pooled, projection)
